# Initial kernel scaffold; baseline (speedup 1.0000x reference)
import jax
import jax.numpy as jnp
from jax.experimental import pallas as pl


def kernel(target_probs, bonus_token_ids, draft_token_ids):
    raise NotImplementedError("write your pallas kernel here")


# speedup vs baseline: 1.1848x; 1.1848x over previous
import functools

import jax
import jax.numpy as jnp
from jax.experimental import pallas as pl
from jax.experimental.pallas import tpu as pltpu

_B, _K, _V = 64, 8, 100000
_R = _B * _K
_VC = 4096
_NJ = (_V + _VC - 1) // _VC

_POSTERIOR_THRESHOLD = 0.3
_POSTERIOR_ALPHA = 0.09


def _stream_body(tp_ref, idx_ref, ent_ref, cand_ref, midx_ref, mval_ref):
    j = pl.program_id(0)
    p = tp_ref[...]
    gcol = jax.lax.broadcasted_iota(jnp.int32, (_R, _VC), 1) + j * _VC
    valid = gcol < _V

    ent_part = jnp.sum(jnp.where(valid, p * jnp.log(p + 1e-5), 0.0),
                       axis=1, keepdims=True)
    cand_part = jnp.sum(jnp.where(gcol == idx_ref[...], p, 0.0),
                        axis=1, keepdims=True)

    pm = jnp.where(valid, p, -jnp.inf)
    cmax = jnp.max(pm, axis=1, keepdims=True)
    cidx = jnp.min(jnp.where(pm == cmax, gcol, _V), axis=1, keepdims=True)

    @pl.when(j == 0)
    def _():
        ent_ref[...] = ent_part
        cand_ref[...] = cand_part
        mval_ref[...] = cmax
        midx_ref[...] = cidx

    @pl.when(j > 0)
    def _():
        ent_ref[...] += ent_part
        cand_ref[...] += cand_part
        better = cmax > mval_ref[...]
        midx_ref[...] = jnp.where(better, cidx, midx_ref[...])
        mval_ref[...] = jnp.where(better, cmax, mval_ref[...])


def _assemble_body(ent_ref, cand_ref, midx_ref, draft_ref, bonus_ref, out_ref):
    ent = -ent_ref[...]
    thr = jnp.minimum(jnp.full_like(ent, _POSTERIOR_THRESHOLD),
                      jnp.exp(-ent) * _POSTERIOR_ALPHA)
    accepted = cand_ref[...] > thr
    k_iota = jax.lax.broadcasted_iota(jnp.int32, (_B, _K), 1)
    limits = jnp.min(jnp.where(~accepted, k_iota, _K), axis=1, keepdims=True)

    accepted_mask = k_iota < limits
    after = k_iota == limits
    draft = draft_ref[...]
    out = jnp.where(accepted_mask, draft, -1)
    recovered = jnp.where(k_iota == 0, midx_ref[...], -1)
    out = jnp.where(after, recovered, out)
    bonus_col = jnp.where(limits == _K, bonus_ref[...], -1)
    out_ref[:, 0:_K] = out
    out_ref[:, _K:_K + 1] = bonus_col


@jax.jit
def kernel(target_probs, bonus_token_ids, draft_token_ids):
    tp = target_probs.reshape(_R, _V)
    draft = draft_token_ids.astype(jnp.int32)
    idx = draft.reshape(_R, 1)

    ent, cand, midx, _ = pl.pallas_call(
        _stream_body,
        grid=(_NJ,),
        in_specs=[
            pl.BlockSpec((_R, _VC), lambda j: (0, j)),
            pl.BlockSpec((_R, 1), lambda j: (0, 0)),
        ],
        out_specs=[
            pl.BlockSpec((_R, 1), lambda j: (0, 0)),
            pl.BlockSpec((_R, 1), lambda j: (0, 0)),
            pl.BlockSpec((_R, 1), lambda j: (0, 0)),
            pl.BlockSpec((_R, 1), lambda j: (0, 0)),
        ],
        out_shape=[
            jax.ShapeDtypeStruct((_R, 1), jnp.float32),
            jax.ShapeDtypeStruct((_R, 1), jnp.float32),
            jax.ShapeDtypeStruct((_R, 1), jnp.int32),
            jax.ShapeDtypeStruct((_R, 1), jnp.float32),
        ],
    )(tp, idx)

    ent = ent.reshape(_B, _K)
    cand = cand.reshape(_B, _K)
    midx = midx.reshape(_B, _K)

    out = pl.pallas_call(
        _assemble_body,
        out_shape=jax.ShapeDtypeStruct((_B, _K + 1), jnp.int32),
    )(ent, cand, midx, draft, bonus_token_ids.astype(jnp.int32))
    return out
